# initial kernel scaffold (unmeasured)
import jax
import jax.numpy as jnp
from jax import lax
from jax.experimental import pallas as pl
from jax.experimental.pallas import tpu as pltpu

N_DEV = 32


def kernel(x, w_mat):
    m_glob, k_per = x.shape
    k_glob, n = w_mat.shape
    m_per = m_glob // N_DEV

    def body(x_ref, w_ref, out_ref, xg_ref, send_sem, recv_sems):
        my = lax.axis_index("i")

        xg_ref[:, pl.ds(my * k_per, k_per)] = x_ref[pl.ds(my * m_per, m_per), :]

        for d in range(1, N_DEV):
            tgt = lax.rem(my + d, N_DEV)
            rdma = pltpu.make_async_remote_copy(
                src_ref=x_ref.at[pl.ds(tgt * m_per, m_per), :],
                dst_ref=xg_ref.at[:, pl.ds(my * k_per, k_per)],
                send_sem=send_sem,
                recv_sem=recv_sems.at[my],
                device_id=(tgt,),
                device_id_type=pl.DeviceIdType.MESH,
            )
            rdma.start()
            rdma.wait_send()

        for d in range(1, N_DEV):
            src = lax.rem(my - d + N_DEV, N_DEV)
            recv = pltpu.make_async_remote_copy(
                src_ref=x_ref.at[pl.ds(0, m_per), :],
                dst_ref=xg_ref.at[:, pl.ds(src * k_per, k_per)],
                send_sem=send_sem,
                recv_sem=recv_sems.at[src],
                device_id=(my,),
                device_id_type=pl.DeviceIdType.MESH,
            )
            recv.wait_recv()

        out_ref[:, :] = jnp.dot(
            xg_ref[:, :], w_ref[:, :], preferred_element_type=jnp.float32
        )

    return pl.pallas_call(
        body,
        out_shape=jax.ShapeDtypeStruct((m_per, n), jnp.float32),
        in_specs=[
            pl.BlockSpec(memory_space=pltpu.VMEM),
            pl.BlockSpec(memory_space=pltpu.VMEM),
        ],
        out_specs=pl.BlockSpec(memory_space=pltpu.VMEM),
        scratch_shapes=[
            pltpu.VMEM((m_per, k_glob), jnp.float32),
            pltpu.SemaphoreType.DMA,
            pltpu.SemaphoreType.DMA((N_DEV,)),
        ],
        compiler_params=pltpu.CompilerParams(collective_id=0),
    )(x, w_mat)


# baseline (device time: 43350 ns/iter reference)
import jax
import jax.numpy as jnp
from jax import lax
from jax.experimental import pallas as pl
from jax.experimental.pallas import tpu as pltpu

N_DEV = 32


def kernel(x, w_mat):
    m_glob, k_per = x.shape
    k_glob, n = w_mat.shape
    m_per = m_glob // N_DEV

    def body(x_ref, w_ref, out_ref, xg_ref, send_sem, recv_sems):
        my = lax.axis_index("i")

        xg_ref[my] = x_ref[pl.ds(my * m_per, m_per), :]

        for d in range(1, N_DEV):
            tgt = lax.rem(my + d, N_DEV)
            rdma = pltpu.make_async_remote_copy(
                src_ref=x_ref.at[pl.ds(tgt * m_per, m_per), :],
                dst_ref=xg_ref.at[my],
                send_sem=send_sem,
                recv_sem=recv_sems.at[my],
                device_id=(tgt,),
                device_id_type=pl.DeviceIdType.MESH,
            )
            rdma.start()
            rdma.wait_send()

        for d in range(1, N_DEV):
            src = lax.rem(my - d + N_DEV, N_DEV)
            recv = pltpu.make_async_remote_copy(
                src_ref=x_ref.at[pl.ds(0, m_per), :],
                dst_ref=xg_ref.at[src],
                send_sem=send_sem,
                recv_sem=recv_sems.at[src],
                device_id=(my,),
                device_id_type=pl.DeviceIdType.MESH,
            )
            recv.wait_recv()

        acc = jnp.dot(
            xg_ref[0], w_ref[0:k_per, :], preferred_element_type=jnp.float32
        )
        for j in range(1, N_DEV):
            acc = acc + jnp.dot(
                xg_ref[j],
                w_ref[j * k_per : (j + 1) * k_per, :],
                preferred_element_type=jnp.float32,
            )
        out_ref[:, :] = acc

    return pl.pallas_call(
        body,
        out_shape=jax.ShapeDtypeStruct((m_per, n), jnp.float32),
        in_specs=[
            pl.BlockSpec(memory_space=pltpu.VMEM),
            pl.BlockSpec(memory_space=pltpu.VMEM),
        ],
        out_specs=pl.BlockSpec(memory_space=pltpu.VMEM),
        scratch_shapes=[
            pltpu.VMEM((N_DEV, m_per, k_per), jnp.float32),
            pltpu.SemaphoreType.DMA,
            pltpu.SemaphoreType.DMA((N_DEV,)),
        ],
    )(x, w_mat)


# device time: 24531 ns/iter; 1.7672x vs baseline; 1.7672x over previous
import jax
import jax.numpy as jnp
from jax import lax
from jax.experimental import pallas as pl
from jax.experimental.pallas import tpu as pltpu

N_DEV = 32


def kernel(x, w_mat):
    m_glob, k_per = x.shape
    k_glob, n = w_mat.shape
    m_per = m_glob // N_DEV

    def body(x_ref, w_ref, out_ref, xg_ref, send_sems, recv_sems):
        my = lax.axis_index("i")

        sends = []
        for d in range(1, N_DEV):
            tgt = lax.rem(my + d, N_DEV)
            rdma = pltpu.make_async_remote_copy(
                src_ref=x_ref.at[pl.ds(tgt * m_per, m_per), :],
                dst_ref=xg_ref.at[my],
                send_sem=send_sems.at[d],
                recv_sem=recv_sems.at[my],
                device_id=(tgt,),
                device_id_type=pl.DeviceIdType.MESH,
            )
            rdma.start()
            sends.append(rdma)

        xg_ref[my] = x_ref[pl.ds(my * m_per, m_per), :]

        acc = None
        for j in range(N_DEV):
            recv = pltpu.make_async_remote_copy(
                src_ref=x_ref.at[pl.ds(0, m_per), :],
                dst_ref=xg_ref.at[j],
                send_sem=send_sems.at[0],
                recv_sem=recv_sems.at[j],
                device_id=(my,),
                device_id_type=pl.DeviceIdType.MESH,
            )

            @pl.when(j != my)
            def _():
                recv.wait_recv()

            term = jnp.dot(
                xg_ref[j],
                w_ref[j * k_per : (j + 1) * k_per, :],
                preferred_element_type=jnp.float32,
            )
            acc = term if acc is None else acc + term
        out_ref[:, :] = acc

        for rdma in sends:
            rdma.wait_send()

    return pl.pallas_call(
        body,
        out_shape=jax.ShapeDtypeStruct((m_per, n), jnp.float32),
        in_specs=[
            pl.BlockSpec(memory_space=pltpu.VMEM),
            pl.BlockSpec(memory_space=pltpu.VMEM),
        ],
        out_specs=pl.BlockSpec(memory_space=pltpu.VMEM),
        scratch_shapes=[
            pltpu.VMEM((N_DEV, m_per, k_per), jnp.float32),
            pltpu.SemaphoreType.DMA((N_DEV,)),
            pltpu.SemaphoreType.DMA((N_DEV,)),
        ],
    )(x, w_mat)
